# baseline (speedup 1.0000x reference)
import jax
import jax.numpy as jnp
from jax.experimental import pallas as pl

_GS = 64
_PARAM = 128
_KNN = 8
_FLAG = 7
_NUMC = 10000.0
_EMPTY = 100000000.0
_SELF_C = 100.0

_MB = 2


def _round_bf16(v):
    return v.astype(jnp.bfloat16).astype(jnp.float32)


def _body(x_ref, f_ref, bmat_ref, diag_ref, out_ref):
    xb = _round_bf16(x_ref[...])
    frow = _round_bf16(f_ref[...])
    fsub = xb[:, :, _FLAG:_FLAG + 1]
    isval = fsub * frow

    diff = xb[:, :, None, :] - xb[:, None, :, :]
    dsq = (diff * diff).astype(jnp.bfloat16)
    delt = jnp.dot(
        dsq.reshape(_MB * _GS, _GS * _PARAM), bmat_ref[...],
        preferred_element_type=jnp.float32,
    ).reshape(_MB, _GS, _GS)

    ji = jax.lax.broadcasted_iota(jnp.int32, (_MB, _GS, _GS), 1)
    d = delt + (1.0 - isval) * _EMPTY
    d = d + diag_ref[...]

    dm = d
    big = jnp.int32(1 << 30)
    for _ in range(_KNN - 1):
        m = jnp.min(dm, axis=1, keepdims=True)
        jidx = jnp.where(dm == m, ji, big)
        amin = jnp.min(jidx, axis=1, keepdims=True)
        dm = jnp.where(ji == amin, jnp.float32(jnp.inf), dm)
    kth = jnp.min(dm, axis=1, keepdims=True)

    su = d - kth
    t = jnp.float32(_NUMC) * su
    rel = jnp.maximum(1.0 - t, 0.0) - jnp.maximum(-t, 0.0)
    rel = jnp.maximum(rel, 0.0) - jnp.maximum(rel - 1.0, 0.0)
    dez2 = rel * isval
    numnei = jnp.sum(dez2, axis=1, keepdims=True)
    factor = jnp.float32(float(_KNN)) / (numnei + 1e-11)
    out_ref[...] = jnp.transpose(dez2 * factor, (0, 2, 1))


def kernel(x, metrik):
    b = x.shape[0]
    f3 = x[:, :, _FLAG].reshape(b, 1, _GS)
    eye = jnp.eye(_GS, dtype=jnp.float32)
    bmat = (eye[:, None, :] * metrik.reshape(1, _PARAM, 1)).reshape(
        _GS * _PARAM, _GS).astype(jnp.bfloat16)
    diag = (_SELF_C * eye).reshape(1, _GS, _GS)
    rel = pl.pallas_call(
        _body,
        grid=(b // _MB,),
        in_specs=[
            pl.BlockSpec((_MB, _GS, _PARAM), lambda i: (i, 0, 0)),
            pl.BlockSpec((_MB, 1, _GS), lambda i: (i, 0, 0)),
            pl.BlockSpec((_GS * _PARAM, _GS), lambda i: (0, 0)),
            pl.BlockSpec((1, _GS, _GS), lambda i: (0, 0, 0)),
        ],
        out_specs=pl.BlockSpec((_MB, _GS, _GS), lambda i: (i, 0, 0)),
        out_shape=jax.ShapeDtypeStruct((b, _GS, _GS), jnp.float32),
    )(x, f3, bmat, diag)
    return (rel, x)


# speedup vs baseline: 1.2002x; 1.2002x over previous
import jax
import jax.numpy as jnp
from jax.experimental import pallas as pl

_GS = 64
_PARAM = 128
_KNN = 8
_FLAG = 7
_NUMC = 10000.0
_EMPTY = 100000000.0
_SELF_C = 100.0

_MB = 8


def _round_bf16(v):
    return v.astype(jnp.bfloat16).astype(jnp.float32)


def _body(x_ref, f_ref, bmat_ref, diag_ref, out_ref):
    xb = _round_bf16(x_ref[...])
    frow = _round_bf16(f_ref[...])
    fsub = xb[:, :, _FLAG:_FLAG + 1]
    isval = fsub * frow

    diff = xb[:, :, None, :] - xb[:, None, :, :]
    dsq = (diff * diff).astype(jnp.bfloat16)
    delt = jnp.dot(
        dsq.reshape(_MB * _GS, _GS * _PARAM), bmat_ref[...],
        preferred_element_type=jnp.float32,
    ).reshape(_MB, _GS, _GS)

    ji = jax.lax.broadcasted_iota(jnp.int32, (_MB, _GS, _GS), 1)
    d = delt + (1.0 - isval) * _EMPTY
    d = d + diag_ref[...]

    dm = d
    big = jnp.int32(1 << 30)
    for _ in range(_KNN - 1):
        m = jnp.min(dm, axis=1, keepdims=True)
        jidx = jnp.where(dm == m, ji, big)
        amin = jnp.min(jidx, axis=1, keepdims=True)
        dm = jnp.where(ji == amin, jnp.float32(jnp.inf), dm)
    kth = jnp.min(dm, axis=1, keepdims=True)

    su = d - kth
    t = jnp.float32(_NUMC) * su
    rel = jnp.maximum(1.0 - t, 0.0) - jnp.maximum(-t, 0.0)
    rel = jnp.maximum(rel, 0.0) - jnp.maximum(rel - 1.0, 0.0)
    dez2 = rel * isval
    numnei = jnp.sum(dez2, axis=1, keepdims=True)
    factor = jnp.float32(float(_KNN)) / (numnei + 1e-11)
    out_ref[...] = jnp.transpose(dez2 * factor, (0, 2, 1))


def kernel(x, metrik):
    b = x.shape[0]
    f3 = x[:, :, _FLAG].reshape(b, 1, _GS)
    eye = jnp.eye(_GS, dtype=jnp.float32)
    bmat = (eye[:, None, :] * metrik.reshape(1, _PARAM, 1)).reshape(
        _GS * _PARAM, _GS).astype(jnp.bfloat16)
    diag = (_SELF_C * eye).reshape(1, _GS, _GS)
    rel = pl.pallas_call(
        _body,
        grid=(b // _MB,),
        in_specs=[
            pl.BlockSpec((_MB, _GS, _PARAM), lambda i: (i, 0, 0)),
            pl.BlockSpec((_MB, 1, _GS), lambda i: (i, 0, 0)),
            pl.BlockSpec((_GS * _PARAM, _GS), lambda i: (0, 0)),
            pl.BlockSpec((1, _GS, _GS), lambda i: (0, 0, 0)),
        ],
        out_specs=pl.BlockSpec((_MB, _GS, _GS), lambda i: (i, 0, 0)),
        out_shape=jax.ShapeDtypeStruct((b, _GS, _GS), jnp.float32),
    )(x, f3, bmat, diag)
    return (rel, x)


# speedup vs baseline: 1.2114x; 1.0093x over previous
import jax
import jax.numpy as jnp
from jax.experimental import pallas as pl

_GS = 64
_PARAM = 128
_KNN = 8
_FLAG = 7
_NUMC = 10000.0
_EMPTY = 100000000.0
_SELF_C = 100.0

_MB = 16


def _round_bf16(v):
    return v.astype(jnp.bfloat16).astype(jnp.float32)


def _body(x_ref, f_ref, bmat_ref, diag_ref, out_ref):
    xb = _round_bf16(x_ref[...])
    frow = _round_bf16(f_ref[...])
    fsub = xb[:, :, _FLAG:_FLAG + 1]
    isval = fsub * frow

    diff = xb[:, :, None, :] - xb[:, None, :, :]
    dsq = (diff * diff).astype(jnp.bfloat16)
    delt = jnp.dot(
        dsq.reshape(_MB * _GS, _GS * _PARAM), bmat_ref[...],
        preferred_element_type=jnp.float32,
    ).reshape(_MB, _GS, _GS)

    ji = jax.lax.broadcasted_iota(jnp.int32, (_MB, _GS, _GS), 1)
    d = delt + (1.0 - isval) * _EMPTY
    d = d + diag_ref[...]

    dm = d
    big = jnp.int32(1 << 30)
    for _ in range(_KNN - 1):
        m = jnp.min(dm, axis=1, keepdims=True)
        jidx = jnp.where(dm == m, ji, big)
        amin = jnp.min(jidx, axis=1, keepdims=True)
        dm = jnp.where(ji == amin, jnp.float32(jnp.inf), dm)
    kth = jnp.min(dm, axis=1, keepdims=True)

    su = d - kth
    t = jnp.float32(_NUMC) * su
    rel = jnp.maximum(1.0 - t, 0.0) - jnp.maximum(-t, 0.0)
    rel = jnp.maximum(rel, 0.0) - jnp.maximum(rel - 1.0, 0.0)
    dez2 = rel * isval
    numnei = jnp.sum(dez2, axis=1, keepdims=True)
    factor = jnp.float32(float(_KNN)) / (numnei + 1e-11)
    out_ref[...] = jnp.transpose(dez2 * factor, (0, 2, 1))


def kernel(x, metrik):
    b = x.shape[0]
    f3 = x[:, :, _FLAG].reshape(b, 1, _GS)
    eye = jnp.eye(_GS, dtype=jnp.float32)
    bmat = (eye[:, None, :] * metrik.reshape(1, _PARAM, 1)).reshape(
        _GS * _PARAM, _GS).astype(jnp.bfloat16)
    diag = (_SELF_C * eye).reshape(1, _GS, _GS)
    rel = pl.pallas_call(
        _body,
        grid=(b // _MB,),
        in_specs=[
            pl.BlockSpec((_MB, _GS, _PARAM), lambda i: (i, 0, 0)),
            pl.BlockSpec((_MB, 1, _GS), lambda i: (i, 0, 0)),
            pl.BlockSpec((_GS * _PARAM, _GS), lambda i: (0, 0)),
            pl.BlockSpec((1, _GS, _GS), lambda i: (0, 0, 0)),
        ],
        out_specs=pl.BlockSpec((_MB, _GS, _GS), lambda i: (i, 0, 0)),
        out_shape=jax.ShapeDtypeStruct((b, _GS, _GS), jnp.float32),
    )(x, f3, bmat, diag)
    return (rel, x)
